# baseline (device time: 42479 ns/iter reference)
import jax
import jax.numpy as jnp
from jax import lax
from jax.experimental import pallas as pl
from jax.experimental.pallas import tpu as pltpu

N_DEV = 32


def kernel(dy, W):
    m, _ = dy.shape
    n = W.shape[0]
    rows = m // N_DEV

    def body(dy_ref, w_ref, out_ref, p_ref, rs_buf, ag_buf,
             send1, recv1, send2, recv2):
        me = lax.axis_index("i")

        barrier_sem = pltpu.get_barrier_semaphore()
        for off in range(1, N_DEV):
            pl.semaphore_signal(
                barrier_sem, inc=1,
                device_id=lax.rem(me + off, N_DEV),
                device_id_type=pl.DeviceIdType.LOGICAL,
            )
        pl.semaphore_wait(barrier_sem, N_DEV - 1)

        p_ref[:, :] = lax.dot_general(
            dy_ref[:, :], w_ref[:, :],
            dimension_numbers=(((1,), (1,)), ((), ())),
            preferred_element_type=jnp.float32,
        )

        sends1 = []
        for off in range(1, N_DEV):
            tgt = lax.rem(me + off, N_DEV)
            rdma = pltpu.make_async_remote_copy(
                src_ref=p_ref.at[pl.ds(tgt * rows, rows), :],
                dst_ref=rs_buf.at[me],
                send_sem=send1.at[off],
                recv_sem=recv1.at[me],
                device_id=tgt,
                device_id_type=pl.DeviceIdType.LOGICAL,
            )
            rdma.start()
            sends1.append(rdma)

        rs_buf[pl.ds(me, 1), :, :] = p_ref[
            pl.ds(me * rows, rows), :
        ].reshape(1, rows, n)

        for off in range(1, N_DEV):
            src = lax.rem(me + off, N_DEV)
            pltpu.make_async_remote_copy(
                src_ref=rs_buf.at[src],
                dst_ref=rs_buf.at[src],
                send_sem=send1.at[off],
                recv_sem=recv1.at[src],
                device_id=me,
                device_id_type=pl.DeviceIdType.LOGICAL,
            ).wait_recv()

        ag_buf[pl.ds(me, 1), :, :] = jnp.sum(
            rs_buf[:, :, :], axis=0, keepdims=True
        )

        sends2 = []
        for off in range(1, N_DEV):
            tgt = lax.rem(me + off, N_DEV)
            rdma = pltpu.make_async_remote_copy(
                src_ref=ag_buf.at[me],
                dst_ref=ag_buf.at[me],
                send_sem=send2.at[off],
                recv_sem=recv2.at[me],
                device_id=tgt,
                device_id_type=pl.DeviceIdType.LOGICAL,
            )
            rdma.start()
            sends2.append(rdma)

        for off in range(1, N_DEV):
            src = lax.rem(me + off, N_DEV)
            pltpu.make_async_remote_copy(
                src_ref=ag_buf.at[src],
                dst_ref=ag_buf.at[src],
                send_sem=send2.at[off],
                recv_sem=recv2.at[src],
                device_id=me,
                device_id_type=pl.DeviceIdType.LOGICAL,
            ).wait_recv()

        for rdma in sends1 + sends2:
            rdma.wait_send()

        for s in range(N_DEV):
            out_ref[pl.ds(s * rows, rows), :] = ag_buf[s, :, :]

    return pl.pallas_call(
        body,
        out_shape=jax.ShapeDtypeStruct((m, n), jnp.float32),
        in_specs=[
            pl.BlockSpec(memory_space=pltpu.VMEM),
            pl.BlockSpec(memory_space=pltpu.VMEM),
        ],
        out_specs=pl.BlockSpec(memory_space=pltpu.VMEM),
        scratch_shapes=[
            pltpu.VMEM((m, n), jnp.float32),
            pltpu.VMEM((N_DEV, rows, n), jnp.float32),
            pltpu.VMEM((N_DEV, rows, n), jnp.float32),
            pltpu.SemaphoreType.DMA((N_DEV,)),
            pltpu.SemaphoreType.DMA((N_DEV,)),
            pltpu.SemaphoreType.DMA((N_DEV,)),
            pltpu.SemaphoreType.DMA((N_DEV,)),
        ],
        compiler_params=pltpu.CompilerParams(collective_id=0),
    )(dy, W)
